# baseline (speedup 1.0000x reference)
import functools

import jax
import jax.numpy as jnp
from jax import lax
from jax.experimental import pallas as pl
from jax.experimental.pallas import tpu as pltpu
from jax.experimental.pallas import tpu_sc as plsc

NUM_NODES = 1000000
EMBED_DIM = 32
BATCH = 16384

_info = plsc.get_sparse_core_info()
_NC = _info.num_cores
_NS = _info.num_subcores
_NW = _NC * _NS
_B_PER_W = BATCH // _NW
_CHUNK = 128
_NCHUNK = _B_PER_W // _CHUNK

_mesh = plsc.VectorSubcoreMesh(core_axis_name="c", subcore_axis_name="s")


@functools.partial(
    pl.kernel,
    mesh=_mesh,
    out_type=jax.ShapeDtypeStruct((BATCH, EMBED_DIM), jnp.float32),
    scratch_types=[
        pltpu.VMEM((_NCHUNK, _CHUNK), jnp.int32),
        pltpu.VMEM((_B_PER_W, EMBED_DIM), jnp.float32),
        pltpu.SemaphoreType.DMA,
    ],
    compiler_params=pltpu.CompilerParams(use_tc_tiling_on_sc=False),
)
def _gather_kernel(idx_hbm, table_hbm, out_hbm, idx_v, rows_v, sem):
    wid = lax.axis_index("s") * _NC + lax.axis_index("c")
    base = wid * _B_PER_W
    pltpu.sync_copy(idx_hbm.at[pl.ds(wid * _NCHUNK, _NCHUNK)], idx_v)
    copies = []
    for k in range(_NCHUNK):
        copies.append(pltpu.async_copy(
            table_hbm.at[idx_v.at[k]],
            rows_v.at[pl.ds(k * _CHUNK, _CHUNK)], sem))
    for c in copies:
        c.wait()
    pltpu.sync_copy(rows_v, out_hbm.at[pl.ds(base, _B_PER_W)])


def kernel(node_indices, embeddings):
    idx = node_indices.astype(jnp.int32).reshape(_NW * _NCHUNK, _CHUNK)
    return _gather_kernel(idx, embeddings)


# speedup vs baseline: 1.0019x; 1.0019x over previous
import functools

import jax
import jax.numpy as jnp
from jax import lax
from jax.experimental import pallas as pl
from jax.experimental.pallas import tpu as pltpu
from jax.experimental.pallas import tpu_sc as plsc

NUM_NODES = 1000000
EMBED_DIM = 32
BATCH = 16384

_info = plsc.get_sparse_core_info()
_NC = _info.num_cores
_NS = _info.num_subcores
_NW = _NC * _NS
_B_PER_W = BATCH // _NW
_CHUNK = 128
_NCHUNK = _B_PER_W // _CHUNK

_mesh = plsc.VectorSubcoreMesh(core_axis_name="c", subcore_axis_name="s")


@functools.partial(
    pl.kernel,
    mesh=_mesh,
    out_type=jax.ShapeDtypeStruct((BATCH, EMBED_DIM), jnp.float32),
    scratch_types=[
        pltpu.VMEM((_NCHUNK, _CHUNK), jnp.int32),
        pltpu.VMEM((_B_PER_W, EMBED_DIM), jnp.float32),
        pltpu.SemaphoreType.DMA,
    ],
    compiler_params=pltpu.CompilerParams(
        use_tc_tiling_on_sc=False,
        disable_bounds_checks=True,
    ),
)
def _gather_kernel(idx_hbm, table_hbm, out_hbm, idx_v, rows_v, sem):
    wid = lax.axis_index("s") * _NC + lax.axis_index("c")
    base = wid * _B_PER_W
    pltpu.sync_copy(idx_hbm.at[pl.ds(wid * _NCHUNK, _NCHUNK)], idx_v)
    copies = []
    for k in range(_NCHUNK):
        copies.append(pltpu.async_copy(
            table_hbm.at[idx_v.at[k]],
            rows_v.at[pl.ds(k * _CHUNK, _CHUNK)], sem))
    for c in copies:
        c.wait()
    pltpu.sync_copy(rows_v, out_hbm.at[pl.ds(base, _B_PER_W)])


def kernel(node_indices, embeddings):
    idx = node_indices.astype(jnp.int32).reshape(_NW * _NCHUNK, _CHUNK)
    return _gather_kernel(idx, embeddings)


# speedup vs baseline: 1.0039x; 1.0020x over previous
import functools

import jax
import jax.numpy as jnp
from jax import lax
from jax.experimental import pallas as pl
from jax.experimental.pallas import tpu as pltpu
from jax.experimental.pallas import tpu_sc as plsc

NUM_NODES = 1000000
EMBED_DIM = 32
BATCH = 16384

_info = plsc.get_sparse_core_info()
_NC = _info.num_cores
_NS = _info.num_subcores
_NW = _NC * _NS
_B_PER_W = BATCH // _NW
_CHUNK = 128
_NCHUNK = _B_PER_W // _CHUNK

_mesh = plsc.VectorSubcoreMesh(core_axis_name="c", subcore_axis_name="s")


@functools.partial(
    pl.kernel,
    mesh=_mesh,
    out_type=jax.ShapeDtypeStruct((BATCH, EMBED_DIM), jnp.float32),
    scratch_types=[
        pltpu.VMEM((_NCHUNK, _CHUNK), jnp.int32),
        pltpu.VMEM((_B_PER_W, EMBED_DIM), jnp.float32),
        pltpu.SemaphoreType.DMA,
    ],
    compiler_params=pltpu.CompilerParams(
        use_tc_tiling_on_sc=False,
        disable_bounds_checks=True,
        skip_device_barrier=True,
    ),
)
def _gather_kernel(idx_hbm, table_hbm, out_hbm, idx_v, rows_v, sem):
    wid = lax.axis_index("s") * _NC + lax.axis_index("c")
    base = wid * _B_PER_W
    pltpu.sync_copy(idx_hbm.at[pl.ds(wid * _NCHUNK, _NCHUNK)], idx_v)
    copies = []
    for k in range(_NCHUNK):
        copies.append(pltpu.async_copy(
            table_hbm.at[idx_v.at[k]],
            rows_v.at[pl.ds(k * _CHUNK, _CHUNK)], sem))
    for c in copies:
        c.wait()
    pltpu.sync_copy(rows_v, out_hbm.at[pl.ds(base, _B_PER_W)])


def kernel(node_indices, embeddings):
    idx = node_indices.astype(jnp.int32).reshape(_NW * _NCHUNK, _CHUNK)
    return _gather_kernel(idx, embeddings)


# speedup vs baseline: 3.0491x; 3.0373x over previous
import functools

import jax
import jax.numpy as jnp
from jax import lax
from jax.experimental import pallas as pl
from jax.experimental.pallas import tpu as pltpu
from jax.experimental.pallas import tpu_sc as plsc

NUM_NODES = 1000000
EMBED_DIM = 32
BATCH = 16384

_info = plsc.get_sparse_core_info()
_NC = _info.num_cores
_NS = _info.num_subcores
_NW = _NC * _NS
_B_PER_W = BATCH // _NW
_L = 16
_GROUPS = _B_PER_W // _L

_mesh = plsc.VectorSubcoreMesh(core_axis_name="c", subcore_axis_name="s")


@functools.partial(
    pl.kernel,
    mesh=_mesh,
    out_type=jax.ShapeDtypeStruct((BATCH, EMBED_DIM), jnp.float32),
    scratch_types=[
        pltpu.VMEM((_B_PER_W,), jnp.int32),
        pltpu.VMEM((8, EMBED_DIM, 128), jnp.float32),
        pltpu.VMEM((_B_PER_W, EMBED_DIM), jnp.float32),
        pltpu.SemaphoreType.DMA,
    ],
    compiler_params=pltpu.CompilerParams(
        needs_layout_passes=False,
        disable_bounds_checks=True,
    ),
)
def _gather_kernel(idx_hbm, embt_hbm, out_hbm, idx_v, bufs_v, rows_v, sem):
    wid = lax.axis_index("s") * _NC + lax.axis_index("c")
    base = wid * _B_PER_W
    pltpu.sync_copy(idx_hbm.at[pl.ds(base, _B_PER_W)], idx_v)

    d_lo = lax.iota(jnp.int32, _L)
    d_hi = d_lo + _L

    def body(g, carry):
        vec16 = idx_v[pl.ds(g * _L, _L)]
        for half in range(2):
            copies = []
            for l in range(8):
                i = vec16[half * 8 + l]
                c = pl.multiple_of((i >> 7) << 7, 128)
                copies.append(pltpu.async_copy(
                    embt_hbm.at[:, pl.ds(c, 128)], bufs_v.at[l], sem))
            for l in range(8):
                copies[l].wait()
            for l in range(8):
                lane = jnp.full((_L,), vec16[half * 8 + l] & 127, jnp.int32)
                v0 = plsc.load_gather(bufs_v.at[l], [d_lo, lane])
                v1 = plsc.load_gather(bufs_v.at[l], [d_hi, lane])
                j = g * _L + half * 8 + l
                rows_v[j, pl.ds(0, _L)] = v0
                rows_v[j, pl.ds(_L, _L)] = v1
        return carry

    lax.fori_loop(0, _GROUPS, body, 0)
    pltpu.sync_copy(rows_v, out_hbm.at[pl.ds(base, _B_PER_W)])


def kernel(node_indices, embeddings):
    idx = node_indices.astype(jnp.int32)
    return _gather_kernel(idx, embeddings.T)


# speedup vs baseline: 3.1243x; 1.0247x over previous
import functools

import jax
import jax.numpy as jnp
from jax import lax
from jax.experimental import pallas as pl
from jax.experimental.pallas import tpu as pltpu
from jax.experimental.pallas import tpu_sc as plsc

NUM_NODES = 1000000
EMBED_DIM = 32
BATCH = 16384

_info = plsc.get_sparse_core_info()
_NC = _info.num_cores
_NS = _info.num_subcores
_NW = _NC * _NS
_B_PER_W = BATCH // _NW
_L = 16
_GROUPS = _B_PER_W // _L

_mesh = plsc.VectorSubcoreMesh(core_axis_name="c", subcore_axis_name="s")


@functools.partial(
    pl.kernel,
    mesh=_mesh,
    out_type=jax.ShapeDtypeStruct((EMBED_DIM, BATCH), jnp.float32),
    scratch_types=[
        pltpu.VMEM((_B_PER_W,), jnp.int32),
        pltpu.VMEM((8, EMBED_DIM, 128), jnp.float32),
        pltpu.VMEM((EMBED_DIM, _B_PER_W), jnp.float32),
        pltpu.SemaphoreType.DMA,
    ],
    compiler_params=pltpu.CompilerParams(
        needs_layout_passes=False,
        disable_bounds_checks=True,
    ),
)
def _gather_kernel(idx_hbm, embt_hbm, out_hbm, idx_v, bufs_v, rows_v, sem):
    wid = lax.axis_index("s") * _NC + lax.axis_index("c")
    base = wid * _B_PER_W
    pltpu.sync_copy(idx_hbm.at[pl.ds(base, _B_PER_W)], idx_v)

    d_lo = lax.iota(jnp.int32, _L)
    d_hi = d_lo + _L

    def body(g, carry):
        vec16 = idx_v[pl.ds(g * _L, _L)]
        for half in range(2):
            copies = []
            for l in range(8):
                i = vec16[half * 8 + l]
                c = pl.multiple_of((i >> 7) << 7, 128)
                copies.append(pltpu.async_copy(
                    embt_hbm.at[:, pl.ds(c, 128)], bufs_v.at[l], sem))
            for l in range(8):
                copies[l].wait()
            for l in range(8):
                lane = jnp.full((_L,), vec16[half * 8 + l] & 127, jnp.int32)
                v0 = plsc.load_gather(bufs_v.at[l], [d_lo, lane])
                v1 = plsc.load_gather(bufs_v.at[l], [d_hi, lane])
                j = jnp.full((_L,), g * _L + half * 8 + l, jnp.int32)
                plsc.store_scatter(rows_v, [d_lo, j], v0)
                plsc.store_scatter(rows_v, [d_hi, j], v1)
        return carry

    lax.fori_loop(0, _GROUPS, body, 0)
    pltpu.sync_copy(rows_v, out_hbm.at[:, pl.ds(base, _B_PER_W)])


def kernel(node_indices, embeddings):
    idx = node_indices.astype(jnp.int32)
    return _gather_kernel(idx, embeddings.T).T


# speedup vs baseline: 3.8721x; 1.2394x over previous
import functools

import jax
import jax.numpy as jnp
from jax import lax
from jax.experimental import pallas as pl
from jax.experimental.pallas import tpu as pltpu
from jax.experimental.pallas import tpu_sc as plsc

NUM_NODES = 1000000
EMBED_DIM = 32
BATCH = 16384

_info = plsc.get_sparse_core_info()
_NC = _info.num_cores
_NS = _info.num_subcores
_NW = _NC * _NS
_B_PER_W = BATCH // _NW
_L = 16
_GROUPS = _B_PER_W // _L

_mesh = plsc.VectorSubcoreMesh(core_axis_name="c", subcore_axis_name="s")


@functools.partial(
    pl.kernel,
    mesh=_mesh,
    out_type=jax.ShapeDtypeStruct((EMBED_DIM, BATCH), jnp.float32),
    scratch_types=[
        pltpu.VMEM((_B_PER_W,), jnp.int32),
        pltpu.VMEM((16, EMBED_DIM, 128), jnp.float32),
        pltpu.VMEM((EMBED_DIM, _B_PER_W), jnp.float32),
        pltpu.SemaphoreType.DMA,
    ],
    compiler_params=pltpu.CompilerParams(
        needs_layout_passes=False,
        disable_bounds_checks=True,
    ),
)
def _gather_kernel(idx_hbm, embt_hbm, out_hbm, idx_v, bufs_v, rows_v, sem):
    wid = lax.axis_index("s") * _NC + lax.axis_index("c")
    base = wid * _B_PER_W
    pltpu.sync_copy(idx_hbm.at[pl.ds(base, _B_PER_W)], idx_v)

    d_lo = lax.iota(jnp.int32, _L)
    d_hi = d_lo + _L

    def body(g, carry):
        vec16 = idx_v[pl.ds(g * _L, _L)]
        copies = []
        for l in range(_L):
            i = vec16[l]
            c = pl.multiple_of((i >> 7) << 7, 128)
            copies.append(pltpu.async_copy(
                embt_hbm.at[:, pl.ds(c, 128)], bufs_v.at[l], sem))
        for l in range(_L):
            copies[l].wait()
            lane = jnp.full((_L,), vec16[l] & 127, jnp.int32)
            v0 = plsc.load_gather(bufs_v.at[l], [d_lo, lane])
            v1 = plsc.load_gather(bufs_v.at[l], [d_hi, lane])
            j = jnp.full((_L,), g * _L + l, jnp.int32)
            plsc.store_scatter(rows_v, [d_lo, j], v0)
            plsc.store_scatter(rows_v, [d_hi, j], v1)
        return carry

    lax.fori_loop(0, _GROUPS, body, 0)
    pltpu.sync_copy(rows_v, out_hbm.at[:, pl.ds(base, _B_PER_W)])


def kernel(node_indices, embeddings):
    idx = node_indices.astype(jnp.int32)
    return _gather_kernel(idx, embeddings.T).T


# speedup vs baseline: 3.8993x; 1.0070x over previous
import functools

import jax
import jax.numpy as jnp
from jax import lax
from jax.experimental import pallas as pl
from jax.experimental.pallas import tpu as pltpu
from jax.experimental.pallas import tpu_sc as plsc

NUM_NODES = 1000000
EMBED_DIM = 32
BATCH = 16384

_info = plsc.get_sparse_core_info()
_NC = _info.num_cores
_NS = _info.num_subcores
_NW = _NC * _NS
_B_PER_W = BATCH // _NW
_L = 16
_GROUPS = _B_PER_W // _L

_mesh = plsc.VectorSubcoreMesh(core_axis_name="c", subcore_axis_name="s")


@functools.partial(
    pl.kernel,
    mesh=_mesh,
    out_type=jax.ShapeDtypeStruct((EMBED_DIM, BATCH), jnp.float32),
    scratch_types=[
        pltpu.VMEM((_B_PER_W,), jnp.int32),
        pltpu.VMEM((16, EMBED_DIM, 128), jnp.float32),
        pltpu.VMEM((EMBED_DIM, _B_PER_W), jnp.float32),
        pltpu.SemaphoreType.DMA,
    ],
    compiler_params=pltpu.CompilerParams(
        needs_layout_passes=False,
        disable_bounds_checks=True,
    ),
)
def _gather_kernel(idx_hbm, embt_hbm, out_hbm, idx_v, bufs_v, rows_v, sem):
    wid = lax.axis_index("s") * _NC + lax.axis_index("c")
    base = wid * _B_PER_W
    pltpu.sync_copy(idx_hbm.at[pl.ds(base, _B_PER_W)], idx_v)

    slot = lax.iota(jnp.int32, _L)

    def body(g, carry):
        vec16 = idx_v[pl.ds(g * _L, _L)]
        lanes = vec16 & 127
        copies = []
        for l in range(_L):
            i = vec16[l]
            c = pl.multiple_of((i >> 7) << 7, 128)
            copies.append(pltpu.async_copy(
                embt_hbm.at[:, pl.ds(c, 128)], bufs_v.at[l], sem))
        for l in range(_L):
            copies[l].wait()
        for d in range(EMBED_DIM):
            dvec = jnp.full((_L,), d, jnp.int32)
            val = plsc.load_gather(bufs_v, [slot, dvec, lanes])
            rows_v[d, pl.ds(g * _L, _L)] = val
        return carry

    lax.fori_loop(0, _GROUPS, body, 0)
    pltpu.sync_copy(rows_v, out_hbm.at[:, pl.ds(base, _B_PER_W)])


def kernel(node_indices, embeddings):
    idx = node_indices.astype(jnp.int32)
    return _gather_kernel(idx, embeddings.T).T
